# Initial kernel scaffold; baseline (speedup 1.0000x reference)
import jax
import jax.numpy as jnp
from jax.experimental import pallas as pl


def kernel(person_features, other_features, person_boxes, other_boxes, is_person, W, b):
    raise NotImplementedError("write your pallas kernel here")


# speedup vs baseline: 5.8538x; 5.8538x over previous
import functools

import jax
import jax.numpy as jnp
from jax.experimental import pallas as pl


def _relation_kernel(person_ref, other_ref, w_ref, b_ref, out_ref, *,
                     f_num, n_p, n_o, d):
    wp = w_ref[:, :d]
    wo = w_ref[:, d:]
    a = jax.lax.dot_general(person_ref[:], wp, (((1,), (1,)), ((), ())),
                            preferred_element_type=jnp.float32)
    b_mat = jax.lax.dot_general(other_ref[:], wo, (((1,), (1,)), ((), ())),
                                preferred_element_type=jnp.float32)
    b_max = jnp.max(b_mat.reshape(f_num, n_o, d), axis=1)
    b_rep = jnp.broadcast_to(b_max[:, None, :], (f_num, n_p, d))
    out_ref[:] = a + b_rep.reshape(f_num * n_p, d) + b_ref[:]


def kernel(person_features, other_features, person_boxes, other_boxes,
           is_person, W, b):
    f_num, n_p = person_boxes.shape[0], person_boxes.shape[1]
    n_o = other_boxes.shape[1]
    d = person_features.shape[1]
    person = person_features.reshape(f_num * n_p, d)
    other = other_features.reshape(f_num * n_o, d)
    out = pl.pallas_call(
        functools.partial(_relation_kernel, f_num=f_num, n_p=n_p, n_o=n_o, d=d),
        out_shape=jax.ShapeDtypeStruct((f_num * n_p, d), jnp.float32),
    )(person, other, W, b.reshape(1, d))
    return out[:, :, None, None]
